# baseline (speedup 1.0000x reference)
import functools

import jax
import jax.numpy as jnp
from jax import lax
from jax.experimental import pallas as pl
from jax.experimental.pallas import tpu as pltpu
from jax.experimental.pallas import tpu_sc as plsc

D_MODEL = 768
N_EXPERTS = 8
D_FFN = int(D_MODEL * 1.618)
THRESHOLD = 0.8
MAX_K = 4
TOKENS = 2048

TBLK = 256
NT = TOKENS // TBLK


def _router_body(x_ref, wr_ref, xr_ref, logits_ref):
    xb = x_ref[...]
    ms = jnp.mean(xb * xb, axis=-1, keepdims=True)
    xr_ref[...] = xb * jax.lax.rsqrt(ms + 1e-6)
    logits_ref[...] = jnp.dot(xb, wr_ref[...],
                              preferred_element_type=jnp.float32)


_NW = 32
_TPW = TOKENS // _NW
_L = 16


def _sc_gates_body(logits_hbm, gates_hbm, lg_v, gt_v):
    wid = lax.axis_index("s") * 2 + lax.axis_index("c")
    pltpu.sync_copy(logits_hbm.at[wid], lg_v)
    for grp in range(_TPW // _L):
        c0 = grp * _L
        lg = [lg_v[e, pl.ds(c0, _L)] for e in range(N_EXPERTS)]
        mx = lg[0]
        for e in range(1, N_EXPERTS):
            mx = jnp.maximum(mx, lg[e])
        ex = [jnp.exp(v - mx) for v in lg]
        s = ex[0]
        for e in range(1, N_EXPERTS):
            s = s + ex[e]
        inv = 1.0 / s
        p = [v * inv for v in ex]

        rem = [jnp.ones((_L,), jnp.float32) for _ in range(N_EXPERTS)]
        csum = jnp.zeros((_L,), jnp.float32)
        gates = [jnp.zeros((_L,), jnp.float32) for _ in range(N_EXPERTS)]
        for _ in range(MAX_K):
            pm = [rem[e] * (p[e] + 1.0) - 1.0 for e in range(N_EXPERTS)]
            m = pm[0]
            for e in range(1, N_EXPERTS):
                m = jnp.maximum(m, pm[e])
            taken = jnp.zeros((_L,), jnp.float32)
            sel = jnp.where(csum < THRESHOLD, 1.0, 0.0)
            for e in range(N_EXPERTS):
                eq = jnp.where(pm[e] == m, 1.0, 0.0)
                pick = eq * (1.0 - taken)
                taken = taken + pick
                gates[e] = gates[e] + pick * sel * p[e]
                rem[e] = rem[e] * (1.0 - pick)
            csum = csum + m
        for e in range(N_EXPERTS):
            gt_v[e, pl.ds(c0, _L)] = gates[e]
    pltpu.sync_copy(gt_v, gates_hbm.at[wid])


def _sc_gates(logits):
    lg = logits.reshape(_NW, _TPW, N_EXPERTS).transpose(0, 2, 1)
    mesh = plsc.VectorSubcoreMesh(core_axis_name="c", subcore_axis_name="s")
    k = functools.partial(
        pl.kernel,
        mesh=mesh,
        out_type=jax.ShapeDtypeStruct((_NW, N_EXPERTS, _TPW), jnp.float32),
        scratch_types=[
            pltpu.VMEM((N_EXPERTS, _TPW), jnp.float32),
            pltpu.VMEM((N_EXPERTS, _TPW), jnp.float32),
        ],
    )(_sc_gates_body)
    return k(lg).transpose(0, 2, 1).reshape(TOKENS, N_EXPERTS)


def _sigmoid(x):
    return 0.5 * jnp.tanh(0.5 * x) + 0.5


def _expert_body(xr_ref, gates_ref, nw_ref, wd_ref, wu_ref, gw_ref, uw_ref,
                 dw_ref, out_ref, acc_ref):
    e = pl.program_id(0)
    t = pl.program_id(1)
    h32 = xr_ref[...] * nw_ref[0]
    hb = h32.astype(jnp.bfloat16)
    pre = jnp.dot(hb, wd_ref[0], preferred_element_type=jnp.float32)
    sp = pre * _sigmoid(pre)
    g = _sigmoid(jnp.dot(sp.astype(jnp.bfloat16), wu_ref[0],
                         preferred_element_type=jnp.float32))
    a = jnp.dot(hb, gw_ref[0], preferred_element_type=jnp.float32)
    a = a * _sigmoid(a)
    b = jnp.dot(hb, uw_ref[0], preferred_element_type=jnp.float32)
    inner = (a * b).astype(jnp.bfloat16)
    eo = jnp.dot(inner, dw_ref[0], preferred_element_type=jnp.float32) * g

    lane = jax.lax.broadcasted_iota(jnp.int32, (1, N_EXPERTS), 1)
    w = jnp.sum(jnp.where(lane == e, gates_ref[...], 0.0), axis=-1,
                keepdims=True)
    contrib = eo * w
    base = t * TBLK

    @pl.when(e == 0)
    def _():
        acc_ref[pl.ds(base, TBLK), :] = contrib

    @pl.when(e > 0)
    def _():
        acc_ref[pl.ds(base, TBLK), :] = acc_ref[pl.ds(base, TBLK), :] + contrib

    @pl.when(e == N_EXPERTS - 1)
    def _():
        out_ref[...] = acc_ref[pl.ds(base, TBLK), :]


@jax.jit
def kernel(x, W_router, norm_w, wd, wu, gate_w, up_w, down_w):
    xr, logits = pl.pallas_call(
        _router_body,
        grid=(NT,),
        in_specs=[
            pl.BlockSpec((TBLK, D_MODEL), lambda t: (t, 0)),
            pl.BlockSpec((D_MODEL, N_EXPERTS), lambda t: (0, 0)),
        ],
        out_specs=[
            pl.BlockSpec((TBLK, D_MODEL), lambda t: (t, 0)),
            pl.BlockSpec((TBLK, N_EXPERTS), lambda t: (t, 0)),
        ],
        out_shape=[
            jax.ShapeDtypeStruct((TOKENS, D_MODEL), jnp.float32),
            jax.ShapeDtypeStruct((TOKENS, N_EXPERTS), jnp.float32),
        ],
    )(x, W_router)

    gates = _sc_gates(logits)

    wd_b = wd.astype(jnp.bfloat16)
    wu_b = wu.astype(jnp.bfloat16)
    gw_b = gate_w.astype(jnp.bfloat16)
    uw_b = up_w.astype(jnp.bfloat16)
    dw_b = down_w.astype(jnp.bfloat16)

    out = pl.pallas_call(
        _expert_body,
        grid=(N_EXPERTS, NT),
        in_specs=[
            pl.BlockSpec((TBLK, D_MODEL), lambda e, t: (t, 0)),
            pl.BlockSpec((TBLK, N_EXPERTS), lambda e, t: (t, 0)),
            pl.BlockSpec((1, 1, D_MODEL), lambda e, t: (e, 0, 0)),
            pl.BlockSpec((1, D_MODEL, D_MODEL), lambda e, t: (e, 0, 0)),
            pl.BlockSpec((1, D_MODEL, D_MODEL), lambda e, t: (e, 0, 0)),
            pl.BlockSpec((1, D_MODEL, D_FFN), lambda e, t: (e, 0, 0)),
            pl.BlockSpec((1, D_MODEL, D_FFN), lambda e, t: (e, 0, 0)),
            pl.BlockSpec((1, D_FFN, D_MODEL), lambda e, t: (e, 0, 0)),
        ],
        out_specs=pl.BlockSpec((TBLK, D_MODEL), lambda e, t: (t, 0)),
        out_shape=jax.ShapeDtypeStruct((TOKENS, D_MODEL), jnp.float32),
        scratch_shapes=[pltpu.VMEM((TOKENS, D_MODEL), jnp.float32)],
        compiler_params=pltpu.CompilerParams(
            dimension_semantics=("arbitrary", "arbitrary"),
        ),
    )(xr, gates, norm_w.reshape(N_EXPERTS, 1, D_MODEL), wd_b, wu_b, gw_b,
      uw_b, dw_b)
    return out


# speedup vs baseline: 1.1088x; 1.1088x over previous
import functools

import jax
import jax.numpy as jnp
from jax import lax
from jax.experimental import pallas as pl
from jax.experimental.pallas import tpu as pltpu
from jax.experimental.pallas import tpu_sc as plsc

D_MODEL = 768
N_EXPERTS = 8
D_FFN = int(D_MODEL * 1.618)
THRESHOLD = 0.8
MAX_K = 4
TOKENS = 2048

TBLK = 1024
NT = TOKENS // TBLK


def _router_body(x_ref, wr_ref, xr_ref, logits_ref):
    xb = x_ref[...]
    ms = jnp.mean(xb * xb, axis=-1, keepdims=True)
    xr_ref[...] = xb * jax.lax.rsqrt(ms + 1e-6)
    logits_ref[...] = jnp.dot(xb, wr_ref[...],
                              preferred_element_type=jnp.float32)


_NW = 32
_TPW = TOKENS // _NW
_L = 16


def _sc_gates_body(logits_hbm, gates_hbm, lg_v, gt_v):
    wid = lax.axis_index("s") * 2 + lax.axis_index("c")
    pltpu.sync_copy(logits_hbm.at[wid], lg_v)
    for grp in range(_TPW // _L):
        c0 = grp * _L
        lg = [lg_v[e, pl.ds(c0, _L)] for e in range(N_EXPERTS)]
        mx = lg[0]
        for e in range(1, N_EXPERTS):
            mx = jnp.maximum(mx, lg[e])
        ex = [jnp.exp(v - mx) for v in lg]
        s = ex[0]
        for e in range(1, N_EXPERTS):
            s = s + ex[e]
        inv = 1.0 / s
        p = [v * inv for v in ex]

        rem = [jnp.ones((_L,), jnp.float32) for _ in range(N_EXPERTS)]
        csum = jnp.zeros((_L,), jnp.float32)
        gates = [jnp.zeros((_L,), jnp.float32) for _ in range(N_EXPERTS)]
        for _ in range(MAX_K):
            pm = [rem[e] * (p[e] + 1.0) - 1.0 for e in range(N_EXPERTS)]
            m = pm[0]
            for e in range(1, N_EXPERTS):
                m = jnp.maximum(m, pm[e])
            taken = jnp.zeros((_L,), jnp.float32)
            sel = jnp.where(csum < THRESHOLD, 1.0, 0.0)
            for e in range(N_EXPERTS):
                eq = jnp.where(pm[e] == m, 1.0, 0.0)
                pick = eq * (1.0 - taken)
                taken = taken + pick
                gates[e] = gates[e] + pick * sel * p[e]
                rem[e] = rem[e] * (1.0 - pick)
            csum = csum + m
        for e in range(N_EXPERTS):
            gt_v[e, pl.ds(c0, _L)] = gates[e]
    pltpu.sync_copy(gt_v, gates_hbm.at[wid])


def _sc_gates(logits):
    lg = logits.reshape(_NW, _TPW, N_EXPERTS).transpose(0, 2, 1)
    mesh = plsc.VectorSubcoreMesh(core_axis_name="c", subcore_axis_name="s")
    k = functools.partial(
        pl.kernel,
        mesh=mesh,
        out_type=jax.ShapeDtypeStruct((_NW, N_EXPERTS, _TPW), jnp.float32),
        scratch_types=[
            pltpu.VMEM((N_EXPERTS, _TPW), jnp.float32),
            pltpu.VMEM((N_EXPERTS, _TPW), jnp.float32),
        ],
    )(_sc_gates_body)
    return k(lg).transpose(0, 2, 1).reshape(TOKENS, N_EXPERTS)


def _sigmoid(x):
    return 0.5 * jnp.tanh(0.5 * x) + 0.5


def _expert_body(xr_ref, gates_ref, nw_ref, wd_ref, wu_ref, gw_ref, uw_ref,
                 dw_ref, out_ref, acc_ref):
    e = pl.program_id(0)
    t = pl.program_id(1)
    h32 = xr_ref[...] * nw_ref[0]
    hb = h32.astype(jnp.bfloat16)
    pre = jnp.dot(hb, wd_ref[0], preferred_element_type=jnp.float32)
    sp = pre * _sigmoid(pre)
    g = _sigmoid(jnp.dot(sp.astype(jnp.bfloat16), wu_ref[0],
                         preferred_element_type=jnp.float32))
    a = jnp.dot(hb, gw_ref[0], preferred_element_type=jnp.float32)
    a = a * _sigmoid(a)
    b = jnp.dot(hb, uw_ref[0], preferred_element_type=jnp.float32)
    inner = (a * b).astype(jnp.bfloat16)
    eo = jnp.dot(inner, dw_ref[0], preferred_element_type=jnp.float32) * g

    lane = jax.lax.broadcasted_iota(jnp.int32, (1, N_EXPERTS), 1)
    w = jnp.sum(jnp.where(lane == e, gates_ref[...], 0.0), axis=-1,
                keepdims=True)
    contrib = eo * w
    base = t * TBLK

    @pl.when(e == 0)
    def _():
        acc_ref[pl.ds(base, TBLK), :] = contrib

    @pl.when(e > 0)
    def _():
        acc_ref[pl.ds(base, TBLK), :] = acc_ref[pl.ds(base, TBLK), :] + contrib

    @pl.when(e == N_EXPERTS - 1)
    def _():
        out_ref[...] = acc_ref[pl.ds(base, TBLK), :]


@jax.jit
def kernel(x, W_router, norm_w, wd, wu, gate_w, up_w, down_w):
    xr, logits = pl.pallas_call(
        _router_body,
        grid=(NT,),
        in_specs=[
            pl.BlockSpec((TBLK, D_MODEL), lambda t: (t, 0)),
            pl.BlockSpec((D_MODEL, N_EXPERTS), lambda t: (0, 0)),
        ],
        out_specs=[
            pl.BlockSpec((TBLK, D_MODEL), lambda t: (t, 0)),
            pl.BlockSpec((TBLK, N_EXPERTS), lambda t: (t, 0)),
        ],
        out_shape=[
            jax.ShapeDtypeStruct((TOKENS, D_MODEL), jnp.float32),
            jax.ShapeDtypeStruct((TOKENS, N_EXPERTS), jnp.float32),
        ],
    )(x, W_router)

    gates = _sc_gates(logits)

    wd_b = wd.astype(jnp.bfloat16)
    wu_b = wu.astype(jnp.bfloat16)
    gw_b = gate_w.astype(jnp.bfloat16)
    uw_b = up_w.astype(jnp.bfloat16)
    dw_b = down_w.astype(jnp.bfloat16)

    out = pl.pallas_call(
        _expert_body,
        grid=(N_EXPERTS, NT),
        in_specs=[
            pl.BlockSpec((TBLK, D_MODEL), lambda e, t: (t, 0)),
            pl.BlockSpec((TBLK, N_EXPERTS), lambda e, t: (t, 0)),
            pl.BlockSpec((1, 1, D_MODEL), lambda e, t: (e, 0, 0)),
            pl.BlockSpec((1, D_MODEL, D_MODEL), lambda e, t: (e, 0, 0)),
            pl.BlockSpec((1, D_MODEL, D_MODEL), lambda e, t: (e, 0, 0)),
            pl.BlockSpec((1, D_MODEL, D_FFN), lambda e, t: (e, 0, 0)),
            pl.BlockSpec((1, D_MODEL, D_FFN), lambda e, t: (e, 0, 0)),
            pl.BlockSpec((1, D_FFN, D_MODEL), lambda e, t: (e, 0, 0)),
        ],
        out_specs=pl.BlockSpec((TBLK, D_MODEL), lambda e, t: (t, 0)),
        out_shape=jax.ShapeDtypeStruct((TOKENS, D_MODEL), jnp.float32),
        scratch_shapes=[pltpu.VMEM((TOKENS, D_MODEL), jnp.float32)],
        compiler_params=pltpu.CompilerParams(
            dimension_semantics=("arbitrary", "arbitrary"),
        ),
    )(xr, gates, norm_w.reshape(N_EXPERTS, 1, D_MODEL), wd_b, wu_b, gw_b,
      uw_b, dw_b)
    return out


# speedup vs baseline: 1.1179x; 1.0082x over previous
import functools

import jax
import jax.numpy as jnp
from jax import lax
from jax.experimental import pallas as pl
from jax.experimental.pallas import tpu as pltpu
from jax.experimental.pallas import tpu_sc as plsc

D_MODEL = 768
N_EXPERTS = 8
D_FFN = int(D_MODEL * 1.618)
THRESHOLD = 0.8
MAX_K = 4
TOKENS = 2048

TBLK = 1024
NT = TOKENS // TBLK


def _router_body(x_ref, wr_ref, xr_ref, logits_ref):
    xb = x_ref[...]
    ms = jnp.mean(xb * xb, axis=-1, keepdims=True)
    xr_ref[...] = xb * jax.lax.rsqrt(ms + 1e-6)
    logits = jnp.dot(xb, wr_ref[...], preferred_element_type=jnp.float32)
    lt = logits.T
    for w in range(TBLK // 64):
        logits_ref[w] = lt[:, w * 64:(w + 1) * 64]


_NW = 32
_TPW = TOKENS // _NW
_L = 16


def _sc_gates_body(logits_hbm, gates_hbm, lg_v, gt_v):
    wid = lax.axis_index("s") * 2 + lax.axis_index("c")
    pltpu.sync_copy(logits_hbm.at[wid], lg_v)
    for grp in range(_TPW // _L):
        c0 = grp * _L
        lg = [lg_v[e, pl.ds(c0, _L)] for e in range(N_EXPERTS)]
        mx = lg[0]
        for e in range(1, N_EXPERTS):
            mx = jnp.maximum(mx, lg[e])
        ex = [jnp.exp(v - mx) for v in lg]
        s = ex[0]
        for e in range(1, N_EXPERTS):
            s = s + ex[e]
        inv = 1.0 / s
        p = [v * inv for v in ex]

        rem = [jnp.ones((_L,), jnp.float32) for _ in range(N_EXPERTS)]
        csum = jnp.zeros((_L,), jnp.float32)
        gates = [jnp.zeros((_L,), jnp.float32) for _ in range(N_EXPERTS)]
        for _ in range(MAX_K):
            pm = [rem[e] * (p[e] + 1.0) - 1.0 for e in range(N_EXPERTS)]
            m = pm[0]
            for e in range(1, N_EXPERTS):
                m = jnp.maximum(m, pm[e])
            taken = jnp.zeros((_L,), jnp.float32)
            sel = jnp.where(csum < THRESHOLD, 1.0, 0.0)
            for e in range(N_EXPERTS):
                eq = jnp.where(pm[e] == m, 1.0, 0.0)
                pick = eq * (1.0 - taken)
                taken = taken + pick
                gates[e] = gates[e] + pick * sel * p[e]
                rem[e] = rem[e] * (1.0 - pick)
            csum = csum + m
        for e in range(N_EXPERTS):
            gt_v[e, pl.ds(c0, _L)] = gates[e]
    pltpu.sync_copy(gt_v, gates_hbm.at[wid])


def _sc_gates(lg):
    mesh = plsc.VectorSubcoreMesh(core_axis_name="c", subcore_axis_name="s")
    k = functools.partial(
        pl.kernel,
        mesh=mesh,
        out_type=jax.ShapeDtypeStruct((_NW, N_EXPERTS, _TPW), jnp.float32),
        scratch_types=[
            pltpu.VMEM((N_EXPERTS, _TPW), jnp.float32),
            pltpu.VMEM((N_EXPERTS, _TPW), jnp.float32),
        ],
    )(_sc_gates_body)
    return k(lg).transpose(0, 2, 1).reshape(TOKENS, N_EXPERTS)


def _sigmoid(x):
    return 0.5 * jnp.tanh(0.5 * x) + 0.5


def _expert_body(xr_ref, gates_ref, nw_ref, wd_ref, wu_ref, gw_ref, uw_ref,
                 dw_ref, out_ref, acc_ref):
    e = pl.program_id(0)
    t = pl.program_id(1)
    h32 = xr_ref[...] * nw_ref[0]
    hb = h32.astype(jnp.bfloat16)
    pre = jnp.dot(hb, wd_ref[0], preferred_element_type=jnp.float32)
    sp = pre * _sigmoid(pre)
    g = _sigmoid(jnp.dot(sp.astype(jnp.bfloat16), wu_ref[0],
                         preferred_element_type=jnp.float32))
    a = jnp.dot(hb, gw_ref[0], preferred_element_type=jnp.float32)
    a = a * _sigmoid(a)
    b = jnp.dot(hb, uw_ref[0], preferred_element_type=jnp.float32)
    inner = (a * b).astype(jnp.bfloat16)
    eo = jnp.dot(inner, dw_ref[0], preferred_element_type=jnp.float32) * g

    lane = jax.lax.broadcasted_iota(jnp.int32, (1, N_EXPERTS), 1)
    w = jnp.sum(jnp.where(lane == e, gates_ref[...], 0.0), axis=-1,
                keepdims=True)
    contrib = eo * w
    base = t * TBLK

    @pl.when(e == 0)
    def _():
        acc_ref[pl.ds(base, TBLK), :] = contrib

    @pl.when(e > 0)
    def _():
        acc_ref[pl.ds(base, TBLK), :] = acc_ref[pl.ds(base, TBLK), :] + contrib

    @pl.when(e == N_EXPERTS - 1)
    def _():
        out_ref[...] = acc_ref[pl.ds(base, TBLK), :]


@jax.jit
def kernel(x, W_router, norm_w, wd, wu, gate_w, up_w, down_w):
    xr, logits = pl.pallas_call(
        _router_body,
        grid=(NT,),
        in_specs=[
            pl.BlockSpec((TBLK, D_MODEL), lambda t: (t, 0)),
            pl.BlockSpec((D_MODEL, N_EXPERTS), lambda t: (0, 0)),
        ],
        out_specs=[
            pl.BlockSpec((TBLK, D_MODEL), lambda t: (t, 0)),
            pl.BlockSpec((TBLK // 64, N_EXPERTS, 64), lambda t: (t, 0, 0)),
        ],
        out_shape=[
            jax.ShapeDtypeStruct((TOKENS, D_MODEL), jnp.float32),
            jax.ShapeDtypeStruct((_NW, N_EXPERTS, _TPW), jnp.float32),
        ],
    )(x, W_router)

    gates = _sc_gates(logits)

    wd_b = wd.astype(jnp.bfloat16)
    wu_b = wu.astype(jnp.bfloat16)
    gw_b = gate_w.astype(jnp.bfloat16)
    uw_b = up_w.astype(jnp.bfloat16)
    dw_b = down_w.astype(jnp.bfloat16)

    out = pl.pallas_call(
        _expert_body,
        grid=(N_EXPERTS, NT),
        in_specs=[
            pl.BlockSpec((TBLK, D_MODEL), lambda e, t: (t, 0)),
            pl.BlockSpec((TBLK, N_EXPERTS), lambda e, t: (t, 0)),
            pl.BlockSpec((1, 1, D_MODEL), lambda e, t: (e, 0, 0)),
            pl.BlockSpec((1, D_MODEL, D_MODEL), lambda e, t: (e, 0, 0)),
            pl.BlockSpec((1, D_MODEL, D_MODEL), lambda e, t: (e, 0, 0)),
            pl.BlockSpec((1, D_MODEL, D_FFN), lambda e, t: (e, 0, 0)),
            pl.BlockSpec((1, D_MODEL, D_FFN), lambda e, t: (e, 0, 0)),
            pl.BlockSpec((1, D_FFN, D_MODEL), lambda e, t: (e, 0, 0)),
        ],
        out_specs=pl.BlockSpec((TBLK, D_MODEL), lambda e, t: (t, 0)),
        out_shape=jax.ShapeDtypeStruct((TOKENS, D_MODEL), jnp.float32),
        scratch_shapes=[pltpu.VMEM((TOKENS, D_MODEL), jnp.float32)],
        compiler_params=pltpu.CompilerParams(
            dimension_semantics=("arbitrary", "arbitrary"),
        ),
    )(xr, gates, norm_w.reshape(N_EXPERTS, 1, D_MODEL), wd_b, wu_b, gw_b,
      uw_b, dw_b)
    return out
